# baseline (speedup 1.0000x reference)
import functools

import jax
import jax.numpy as jnp
from jax import lax
from jax.experimental import pallas as pl
from jax.experimental.pallas import tpu as pltpu
from jax.experimental.pallas import tpu_sc as plsc

B = 16384
DIM = 64
NC = 2
NS = 16
NW = NC * NS
RPW = B // NW
CHUNK = 128
NCHUNK = RPW // CHUNK
GROUPS = RPW // 16


def _rsqrt(n):
    i = plsc.bitcast(n, jnp.int32)
    y = plsc.bitcast(jnp.int32(0x5F3759DF) - lax.shift_right_logical(i, 1),
                     jnp.float32)
    for _ in range(3):
        y = y * (1.5 - 0.5 * n * y * y)
    return y


@functools.partial(
    pl.kernel,
    mesh=plsc.VectorSubcoreMesh(core_axis_name="c", subcore_axis_name="s"),
    compiler_params=pltpu.CompilerParams(needs_layout_passes=False,
                                         use_tc_tiling_on_sc=False),
    out_type=jax.ShapeDtypeStruct((NW, GROUPS, 16), jnp.float32),
    scratch_types=[
        pltpu.VMEM((NCHUNK, CHUNK), jnp.int32),
        pltpu.VMEM((NCHUNK, CHUNK), jnp.int32),
        pltpu.VMEM((RPW, DIM), jnp.float32),
        pltpu.VMEM((RPW, DIM), jnp.float32),
        pltpu.VMEM((RPW, DIM), jnp.float32),
        pltpu.VMEM((DIM,), jnp.float32),
        pltpu.VMEM((GROUPS, 16), jnp.float32),
        pltpu.SemaphoreType.DMA,
    ],
)
def _bilinear_mf(P_hbm, iw_hbm, il_hbm, q_hbm, w_hbm, out_hbm,
                 iw_v, il_v, win_v, loss_v, q_v, w_v, o_v, sem):
    wid = lax.axis_index("s") * NC + lax.axis_index("c")
    base = wid * RPW

    pltpu.sync_copy(iw_hbm.at[wid], iw_v)
    pltpu.sync_copy(il_hbm.at[wid], il_v)
    pltpu.sync_copy(w_hbm, w_v)
    copies = [pltpu.async_copy(q_hbm.at[pl.ds(base, RPW)], q_v, sem)]
    for j in range(NCHUNK):
        copies.append(pltpu.async_copy(
            P_hbm.at[iw_v.at[j]], win_v.at[pl.ds(j * CHUNK, CHUNK)], sem))
        copies.append(pltpu.async_copy(
            P_hbm.at[il_v.at[j]], loss_v.at[pl.ds(j * CHUNK, CHUNK)], sem))
    for c in copies:
        c.wait()

    iota = lax.iota(jnp.int32, 16)
    zeros = jnp.zeros((16,), jnp.float32)
    w_regs = [w_v[pl.ds(k * 16, 16)] for k in range(DIM // 16)]

    def body(g, carry):
        rows = g * 16 + iota
        nwin, swin, nloss, sloss = zeros, zeros, zeros, zeros
        for d in range(DIM):
            col = jnp.full((16,), d, jnp.int32)
            wv = plsc.load_gather(win_v, [rows, col])
            lv = plsc.load_gather(loss_v, [rows, col])
            qv = plsc.load_gather(q_v, [rows, col])
            u = qv * w_regs[d // 16][d % 16]
            nwin = nwin + wv * wv
            swin = swin + wv * u
            nloss = nloss + lv * lv
            sloss = sloss + lv * u
        sw = jnp.where(nwin == 0.0, 0.0, nwin * _rsqrt(nwin))
        sl = jnp.where(nloss == 0.0, 0.0, nloss * _rsqrt(nloss))
        o_v[g] = (swin / jnp.maximum(sw, 1e-12)
                  - sloss / jnp.maximum(sl, 1e-12))
        return carry

    lax.fori_loop(0, GROUPS, body, 0)
    pltpu.sync_copy(o_v, out_hbm.at[wid])


def kernel(model_win, model_loss, q_emb, P, W_cls):
    iw = model_win.reshape(NW, NCHUNK, CHUNK)
    il = model_loss.reshape(NW, NCHUNK, CHUNK)
    w = W_cls.reshape(DIM)
    out = _bilinear_mf(P, iw, il, q_emb, w)
    return out.reshape(B)
